# baseline (speedup 1.0000x reference)
import functools

import jax
import jax.numpy as jnp
from jax import lax
from jax.experimental import pallas as pl
from jax.experimental.pallas import tpu as pltpu
from jax.experimental.pallas import tpu_sc as plsc

_MESH = plsc.VectorSubcoreMesh(
    core_axis_name="c", subcore_axis_name="s", num_cores=1, num_subcores=1
)


@functools.partial(
    pl.kernel,
    mesh=_MESH,
    out_type=jax.ShapeDtypeStruct((2, 16), jnp.int32),
    scratch_types=[
        pltpu.VMEM((2, 16), jnp.int32),
        pltpu.VMEM((2, 16), jnp.int32),
    ],
    compiler_params=pltpu.CompilerParams(needs_layout_passes=False),
)
def _argsort_desc_sc(lens_hbm, out_hbm, lens_v, idx_v):
    pltpu.sync_copy(lens_hbm, lens_v)
    iota = lax.iota(jnp.int32, 16)
    for row in range(2):
        keys = lens_v[row, :] * 16 + (15 - iota)
        _, idx = plsc.sort_key_val(keys, iota, descending=True)
        idx_v[row, :] = idx
    pltpu.sync_copy(idx_v, out_hbm)


def kernel(sent1, sent2, len1, len2):
    del sent1, sent2
    packed = _argsort_desc_sc(jnp.stack([len1, len2]))
    return (packed[0], packed[1])


# speedup vs baseline: 1.0074x; 1.0074x over previous
import functools

import jax
import jax.numpy as jnp
from jax import lax
from jax.experimental import pallas as pl
from jax.experimental.pallas import tpu as pltpu
from jax.experimental.pallas import tpu_sc as plsc

_MESH = plsc.VectorSubcoreMesh(
    core_axis_name="c", subcore_axis_name="s", num_cores=1, num_subcores=1
)


@functools.partial(
    pl.kernel,
    mesh=_MESH,
    out_type=[
        jax.ShapeDtypeStruct((16,), jnp.int32),
        jax.ShapeDtypeStruct((16,), jnp.int32),
    ],
    scratch_types=[
        pltpu.VMEM((16,), jnp.int32),
        pltpu.VMEM((16,), jnp.int32),
    ],
    compiler_params=pltpu.CompilerParams(needs_layout_passes=False),
)
def _argsort_desc_sc(len1_hbm, len2_hbm, out1_hbm, out2_hbm, lens_v, idx_v):
    def sort_one(len_hbm, out_hbm):
        pltpu.sync_copy(len_hbm, lens_v)
        iota = lax.iota(jnp.int32, 16)
        keys = lens_v[...] * 16 + (15 - iota)
        _, idx = plsc.sort_key_val(keys, iota, descending=True)
        idx_v[...] = idx
        pltpu.sync_copy(idx_v, out_hbm)

    sort_one(len1_hbm, out1_hbm)
    sort_one(len2_hbm, out2_hbm)


def kernel(sent1, sent2, len1, len2):
    del sent1, sent2
    idx1_sort, idx2_sort = _argsort_desc_sc(len1, len2)
    return (idx1_sort, idx2_sort)


# speedup vs baseline: 1.0331x; 1.0255x over previous
import functools

import jax
import jax.numpy as jnp
from jax import lax
from jax.experimental import pallas as pl
from jax.experimental.pallas import tpu as pltpu
from jax.experimental.pallas import tpu_sc as plsc

_MESH = plsc.VectorSubcoreMesh(
    core_axis_name="c", subcore_axis_name="s", num_cores=1, num_subcores=2
)


@functools.partial(
    pl.kernel,
    mesh=_MESH,
    out_type=[
        jax.ShapeDtypeStruct((16,), jnp.int32),
        jax.ShapeDtypeStruct((16,), jnp.int32),
    ],
    scratch_types=[
        pltpu.VMEM((16,), jnp.int32),
        pltpu.VMEM((16,), jnp.int32),
    ],
    compiler_params=pltpu.CompilerParams(needs_layout_passes=False),
)
def _argsort_desc_sc(len1_hbm, len2_hbm, out1_hbm, out2_hbm, lens_v, idx_v):
    def sort_one(len_hbm, out_hbm):
        pltpu.sync_copy(len_hbm, lens_v)
        iota = lax.iota(jnp.int32, 16)
        keys = lens_v[...] * 16 + (15 - iota)
        _, idx = plsc.sort_key_val(keys, iota, descending=True)
        idx_v[...] = idx
        pltpu.sync_copy(idx_v, out_hbm)

    sid = lax.axis_index("s")

    @pl.when(sid == 0)
    def _():
        sort_one(len1_hbm, out1_hbm)

    @pl.when(sid == 1)
    def _():
        sort_one(len2_hbm, out2_hbm)


def kernel(sent1, sent2, len1, len2):
    del sent1, sent2
    idx1_sort, idx2_sort = _argsort_desc_sc(len1, len2)
    return (idx1_sort, idx2_sort)
